# baseline (speedup 1.0000x reference)
import jax
import jax.numpy as jnp
from jax import lax
from jax.experimental import pallas as pl
from jax.experimental.pallas import tpu as pltpu
from jax.experimental.pallas import tpu_sc as plsc

B = 16384
C = 4
E = 64
V = 1000000
NC, NS, L = 2, 16, 16
NW = NC * NS
BPW = B // NW
CB = 64
NCHUNK = BPW // CB
MAXG = 128
W = 2 * E
TCOLS = V // 128
TAIL = V - TCOLS * 128
CPW = 244
XTRA = TCOLS - CPW * NW
NBLK = CPW // 2


def _transpose_body(t_nat, c_nat, t_tl, c_tl, t_out, c_out,
                    inb, outb, colb, si0, si1, so0, so1):
    wid = lax.axis_index("s") * NC + lax.axis_index("c")
    lanes = lax.broadcasted_iota(jnp.int32, (L,), 0)
    sin = (si0, si1)
    sout = (so0, so1)
    start = wid * CPW
    erows = [16 * j + lanes for j in range(4)]

    def transpose_block(ib, ob, ncols):
        def row_body(r, _):
            rr = jnp.full((L,), r, jnp.int32)
            for j in range(4):
                ob[r, pl.ds(16 * j, L)] = plsc.load_gather(ib, [erows[j], rr])
            return 0
        lax.fori_loop(0, ncols, row_body, 0)

    def do_cols(nat, out):
        def issue_in(b, buf):
            c0 = (start + 2 * b) * 128
            return pltpu.async_copy(
                nat.at[:, pl.ds(c0, 256)], inb.at[buf, :, pl.ds(0, 256)],
                sin[buf])

        def wait_in(b, buf):
            c0 = (start + 2 * b) * 128
            pltpu.make_async_copy(
                nat.at[:, pl.ds(c0, 256)], inb.at[buf, :, pl.ds(0, 256)],
                sin[buf]).wait()

        def issue_out(b, buf):
            r0 = (start + 2 * b) * 128
            return pltpu.async_copy(
                outb.at[buf], out.at[pl.ds(r0, 256)], sout[buf])

        def wait_out(b, buf):
            r0 = (start + 2 * b) * 128
            pltpu.make_async_copy(
                outb.at[buf], out.at[pl.ds(r0, 256)], sout[buf]).wait()

        issue_in(jnp.int32(0), 0)
        issue_in(jnp.int32(1), 1)

        def pair_body(m, _):
            for sub in range(2):
                b = 2 * m + sub
                wait_in(b, sub)

                @pl.when(b >= 2)
                def _():
                    wait_out(b - 2, sub)
                transpose_block(inb.at[sub], outb.at[sub], 256)
                issue_out(b, sub)

                @pl.when(b + 2 < NBLK)
                def _():
                    issue_in(b + 2, sub)
            return 0

        lax.fori_loop(0, NBLK // 2, pair_body, 0)
        wait_out(jnp.int32(NBLK - 2), 0)
        wait_out(jnp.int32(NBLK - 1), 1)

    do_cols(t_nat, t_out)
    do_cols(c_nat, c_out)

    def do_leftover(nat, out, tj):
        pltpu.async_copy(
            nat.at[:, pl.ds(tj * 128, 128)], colb.at[:, pl.ds(0, 128)],
            sin[0]).wait()
        transpose_block(colb, outb.at[0], 128)
        pltpu.async_copy(
            outb.at[0, pl.ds(0, 128)], out.at[pl.ds(tj * 128, 128)],
            sout[0]).wait()

    def do_tail(tl, out):
        pltpu.async_copy(tl, outb.at[0, pl.ds(0, TAIL)], sin[0]).wait()
        pltpu.async_copy(
            outb.at[0, pl.ds(0, TAIL)],
            out.at[pl.ds(TCOLS * 128, TAIL)], sout[0]).wait()

    for i in range(XTRA):
        for which in range(2):
            @pl.when(wid == 24 + 2 * i + which)
            def _(i=i, which=which):
                do_leftover(t_nat if which == 0 else c_nat,
                            t_out if which == 0 else c_out,
                            CPW * NW + i)

    @pl.when(wid == 22)
    def _():
        do_tail(t_tl, t_out)

    @pl.when(wid == 23)
    def _():
        do_tail(c_tl, c_out)


def _gather_body(t_idx_hbm, c_idx_hbm, t_tab, c_tab, out_hbm,
                 t_idx_v, c_idx_v, t_rows, c_rows, out_v, sem0, sem1):
    wid = lax.axis_index("s") * NC + lax.axis_index("c")
    base = wid * BPW

    pltpu.sync_copy(t_idx_hbm.at[pl.ds(base, BPW)], t_idx_v)
    pltpu.sync_copy(c_idx_hbm.at[pl.ds(base * C, BPW * C)], c_idx_v)

    sems = (sem0, sem1)

    def issue(g, buf):
        off = g * CB
        cps = [pltpu.async_copy(
            t_tab.at[t_idx_v.at[pl.ds(off, CB)]], t_rows.at[buf], sems[buf])]
        for j in range(CB * C // MAXG):
            cps.append(pltpu.async_copy(
                c_tab.at[c_idx_v.at[pl.ds(off * C + j * MAXG, MAXG)]],
                c_rows.at[buf, pl.ds(j * MAXG, MAXG)], sems[buf]))
        return cps

    lanes = lax.broadcasted_iota(jnp.int32, (L,), 0)
    msk_last = lanes == (L - 1)

    def compute(g, buf):
        tr = t_rows.at[buf]
        cr = c_rows.at[buf]

        def b_body(b, _):
            tv = [tr[b, pl.ds(16 * k, L)] for k in range(4)]
            ib4 = jnp.full((L,), b * C, jnp.int32)
            for c in range(C):
                s = tv[0] * cr[b * C + c, pl.ds(0, L)]
                for k in range(1, 4):
                    s = s + tv[k] * cr[b * C + c, pl.ds(16 * k, L)]
                plsc.store_scatter(out_v, [ib4 + c], plsc.cumsum(s),
                                   mask=msk_last)
            return 0

        lax.fori_loop(0, CB, b_body, 0)
        pltpu.sync_copy(out_v, out_hbm.at[pl.ds((base + g * CB) * C, CB * C)])

    pend = issue(0, 0)
    for g in range(NCHUNK):
        nxt = issue(g + 1, (g + 1) % 2) if g + 1 < NCHUNK else None
        for cp in pend:
            cp.wait()
        compute(g, g % 2)
        pend = nxt


@jax.jit
def kernel(target, context, target_table, context_table):
    if target.ndim == 2:
        target = jnp.squeeze(target, axis=1)
    mesh = plsc.VectorSubcoreMesh(core_axis_name="c", subcore_axis_name="s")
    params = pltpu.CompilerParams(needs_layout_passes=False)

    transpose = pl.kernel(
        _transpose_body,
        out_type=(jax.ShapeDtypeStruct((V, W), jnp.float32),
                  jax.ShapeDtypeStruct((V, W), jnp.float32)),
        mesh=mesh,
        scratch_types=[
            pltpu.VMEM((2, E, 257), jnp.float32),
            pltpu.VMEM((2, 256, W), jnp.float32),
            pltpu.VMEM((E, 129), jnp.float32),
            pltpu.SemaphoreType.DMA,
            pltpu.SemaphoreType.DMA,
            pltpu.SemaphoreType.DMA,
            pltpu.SemaphoreType.DMA,
        ],
        compiler_params=params,
    )
    gather = pl.kernel(
        _gather_body,
        out_type=jax.ShapeDtypeStruct((B * C,), jnp.float32),
        mesh=mesh,
        scratch_types=[
            pltpu.VMEM((BPW,), jnp.int32),
            pltpu.VMEM((BPW * C,), jnp.int32),
            pltpu.VMEM((2, CB, W), jnp.float32),
            pltpu.VMEM((2, CB * C, W), jnp.float32),
            pltpu.VMEM((CB * C,), jnp.float32),
            pltpu.SemaphoreType.DMA,
            pltpu.SemaphoreType.DMA,
        ],
        compiler_params=params,
    )
    t_tl = jnp.pad(target_table[TCOLS * 128:], ((0, 0), (0, E)))
    c_tl = jnp.pad(context_table[TCOLS * 128:], ((0, 0), (0, E)))
    t2, c2 = transpose(jnp.swapaxes(target_table, 0, 1),
                       jnp.swapaxes(context_table, 0, 1), t_tl, c_tl)
    out = gather(target.astype(jnp.int32),
                 context.astype(jnp.int32).reshape(-1), t2, c2)
    return out.reshape(B, C)


# speedup vs baseline: 1.8661x; 1.8661x over previous
import jax
import jax.numpy as jnp
from jax import lax
from jax.experimental import pallas as pl
from jax.experimental.pallas import tpu as pltpu
from jax.experimental.pallas import tpu_sc as plsc

B = 16384
C = 4
E = 64
V = 1000000
NC, NS, L = 2, 16, 16
NW = NC * NS
BPW = B // NW
CB = 64
NCHUNK = BPW // CB
MAXG = 128
W = 2 * E
TCOLS = V // 128
TAIL = V - TCOLS * 128
CPW = 244
XTRA = TCOLS - CPW * NW
NBLK = CPW // 2


def _transpose_body(t_nat, c_nat, t_tl, c_tl, t_out, c_out,
                    inb, outb, colb, si0, si1, so0, so1):
    wid = lax.axis_index("s") * NC + lax.axis_index("c")
    lanes = lax.broadcasted_iota(jnp.int32, (L,), 0)
    sin = (si0, si1)
    sout = (so0, so1)
    start = wid * CPW
    erows = [16 * j + lanes for j in range(4)]

    def transpose_block(ib, ob, ncols):
        @plsc.parallel_loop(0, ncols, unroll=8)
        def _(r):
            rr = jnp.full((L,), r, jnp.int32)
            for j in range(4):
                ob[r, pl.ds(16 * j, L)] = plsc.load_gather(ib, [erows[j], rr])

    def do_cols(nat, out):
        def issue_in(b, buf):
            c0 = (start + 2 * b) * 128
            return pltpu.async_copy(
                nat.at[:, pl.ds(c0, 256)], inb.at[buf, :, pl.ds(0, 256)],
                sin[buf])

        def wait_in(b, buf):
            c0 = (start + 2 * b) * 128
            pltpu.make_async_copy(
                nat.at[:, pl.ds(c0, 256)], inb.at[buf, :, pl.ds(0, 256)],
                sin[buf]).wait()

        def issue_out(b, buf):
            r0 = (start + 2 * b) * 128
            return pltpu.async_copy(
                outb.at[buf], out.at[pl.ds(r0, 256)], sout[buf])

        def wait_out(b, buf):
            r0 = (start + 2 * b) * 128
            pltpu.make_async_copy(
                outb.at[buf], out.at[pl.ds(r0, 256)], sout[buf]).wait()

        issue_in(jnp.int32(0), 0)
        issue_in(jnp.int32(1), 1)

        def pair_body(m, _):
            for sub in range(2):
                b = 2 * m + sub
                wait_in(b, sub)

                @pl.when(b >= 2)
                def _():
                    wait_out(b - 2, sub)
                transpose_block(inb.at[sub], outb.at[sub], 256)
                issue_out(b, sub)

                @pl.when(b + 2 < NBLK)
                def _():
                    issue_in(b + 2, sub)
            return 0

        lax.fori_loop(0, NBLK // 2, pair_body, 0)
        wait_out(jnp.int32(NBLK - 2), 0)
        wait_out(jnp.int32(NBLK - 1), 1)

    do_cols(t_nat, t_out)
    do_cols(c_nat, c_out)

    def do_leftover(nat, out, tj):
        pltpu.async_copy(
            nat.at[:, pl.ds(tj * 128, 128)], colb.at[:, pl.ds(0, 128)],
            sin[0]).wait()
        transpose_block(colb, outb.at[0], 128)
        pltpu.async_copy(
            outb.at[0, pl.ds(0, 128)], out.at[pl.ds(tj * 128, 128)],
            sout[0]).wait()

    def do_tail(tl, out):
        pltpu.async_copy(tl, outb.at[0, pl.ds(0, TAIL)], sin[0]).wait()
        pltpu.async_copy(
            outb.at[0, pl.ds(0, TAIL)],
            out.at[pl.ds(TCOLS * 128, TAIL)], sout[0]).wait()

    for i in range(XTRA):
        for which in range(2):
            @pl.when(wid == 24 + 2 * i + which)
            def _(i=i, which=which):
                do_leftover(t_nat if which == 0 else c_nat,
                            t_out if which == 0 else c_out,
                            CPW * NW + i)

    @pl.when(wid == 22)
    def _():
        do_tail(t_tl, t_out)

    @pl.when(wid == 23)
    def _():
        do_tail(c_tl, c_out)


def _gather_body(t_idx_hbm, c_idx_hbm, t_tab, c_tab, out_hbm,
                 t_idx_v, c_idx_v, t_rows, c_rows, out_v, sem0, sem1):
    wid = lax.axis_index("s") * NC + lax.axis_index("c")
    base = wid * BPW

    pltpu.sync_copy(t_idx_hbm.at[pl.ds(base, BPW)], t_idx_v)
    pltpu.sync_copy(c_idx_hbm.at[pl.ds(base * C, BPW * C)], c_idx_v)

    sems = (sem0, sem1)

    def issue(g, buf):
        off = g * CB
        cps = [pltpu.async_copy(
            t_tab.at[t_idx_v.at[pl.ds(off, CB)]], t_rows.at[buf], sems[buf])]
        for j in range(CB * C // MAXG):
            cps.append(pltpu.async_copy(
                c_tab.at[c_idx_v.at[pl.ds(off * C + j * MAXG, MAXG)]],
                c_rows.at[buf, pl.ds(j * MAXG, MAXG)], sems[buf]))
        return cps

    lanes = lax.broadcasted_iota(jnp.int32, (L,), 0)
    msk_last = lanes == (L - 1)

    def compute(g, buf):
        tr = t_rows.at[buf]
        cr = c_rows.at[buf]

        @plsc.parallel_loop(0, CB, unroll=4)
        def _(b):
            tv = [tr[b, pl.ds(16 * k, L)] for k in range(4)]
            ib4 = jnp.full((L,), b * C, jnp.int32)
            for c in range(C):
                s = tv[0] * cr[b * C + c, pl.ds(0, L)]
                for k in range(1, 4):
                    s = s + tv[k] * cr[b * C + c, pl.ds(16 * k, L)]
                plsc.store_scatter(out_v, [ib4 + c], plsc.cumsum(s),
                                   mask=msk_last)
        pltpu.sync_copy(out_v, out_hbm.at[pl.ds((base + g * CB) * C, CB * C)])

    pend = issue(0, 0)
    for g in range(NCHUNK):
        nxt = issue(g + 1, (g + 1) % 2) if g + 1 < NCHUNK else None
        for cp in pend:
            cp.wait()
        compute(g, g % 2)
        pend = nxt


@jax.jit
def kernel(target, context, target_table, context_table):
    if target.ndim == 2:
        target = jnp.squeeze(target, axis=1)
    mesh = plsc.VectorSubcoreMesh(core_axis_name="c", subcore_axis_name="s")
    params = pltpu.CompilerParams(needs_layout_passes=False)

    transpose = pl.kernel(
        _transpose_body,
        out_type=(jax.ShapeDtypeStruct((V, W), jnp.float32),
                  jax.ShapeDtypeStruct((V, W), jnp.float32)),
        mesh=mesh,
        scratch_types=[
            pltpu.VMEM((2, E, 257), jnp.float32),
            pltpu.VMEM((2, 256, W), jnp.float32),
            pltpu.VMEM((E, 129), jnp.float32),
            pltpu.SemaphoreType.DMA,
            pltpu.SemaphoreType.DMA,
            pltpu.SemaphoreType.DMA,
            pltpu.SemaphoreType.DMA,
        ],
        compiler_params=params,
    )
    gather = pl.kernel(
        _gather_body,
        out_type=jax.ShapeDtypeStruct((B * C,), jnp.float32),
        mesh=mesh,
        scratch_types=[
            pltpu.VMEM((BPW,), jnp.int32),
            pltpu.VMEM((BPW * C,), jnp.int32),
            pltpu.VMEM((2, CB, W), jnp.float32),
            pltpu.VMEM((2, CB * C, W), jnp.float32),
            pltpu.VMEM((CB * C,), jnp.float32),
            pltpu.SemaphoreType.DMA,
            pltpu.SemaphoreType.DMA,
        ],
        compiler_params=params,
    )
    t_tl = jnp.pad(target_table[TCOLS * 128:], ((0, 0), (0, E)))
    c_tl = jnp.pad(context_table[TCOLS * 128:], ((0, 0), (0, E)))
    t2, c2 = transpose(jnp.swapaxes(target_table, 0, 1),
                       jnp.swapaxes(context_table, 0, 1), t_tl, c_tl)
    out = gather(target.astype(jnp.int32),
                 context.astype(jnp.int32).reshape(-1), t2, c2)
    return out.reshape(B, C)


# speedup vs baseline: 4.8726x; 2.6111x over previous
import jax
import jax.numpy as jnp
from jax import lax
from jax.experimental import pallas as pl
from jax.experimental.pallas import tpu as pltpu
from jax.experimental.pallas import tpu_sc as plsc

B = 16384
C = 4
E = 64
V = 1000000
NC, NS, L = 2, 16, 16
NW = NC * NS
BPW = B // NW
CB = 64
NCHUNK = BPW // CB
MAXG = 128
W = 2 * E
TCOLS = V // 128
TAIL = V - TCOLS * 128
CPW = 244
XTRA = TCOLS - CPW * NW
NBLK = CPW // 2


def _transpose_body(t_nat, c_nat, t_tl, c_tl, t_out, c_out,
                    inb, outb, colb, si0, si1, so0, so1):
    wid = lax.axis_index("s") * NC + lax.axis_index("c")
    lanes = lax.broadcasted_iota(jnp.int32, (L,), 0)
    sin = (si0, si1)
    sout = (so0, so1)
    start = wid * CPW
    erows = [16 * j + lanes for j in range(4)]

    def transpose_block(ib, ob, ncols):
        @plsc.parallel_loop(0, (ncols // L) * 4, unroll=1)
        def _(m):
            r0 = lax.shift_right_logical(m, 2) * L
            erow = (m & 3) * L + lanes
            for s in range(L):
                colv = r0 + ((lanes + s) & (L - 1))
                v = plsc.load_gather(ib, [erow, colv])
                plsc.store_scatter(ob, [colv, erow], v)

    def do_cols(nat, out):
        def issue_in(b, buf):
            c0 = (start + 2 * b) * 128
            pltpu.async_copy(
                nat.at[:, pl.ds(c0, 256)], inb.at[buf], sin[buf])

        def wait_in(b, buf):
            c0 = (start + 2 * b) * 128
            pltpu.make_async_copy(
                nat.at[:, pl.ds(c0, 256)], inb.at[buf], sin[buf]).wait()

        def issue_out(b, buf):
            r0 = (start + 2 * b) * 128
            return pltpu.async_copy(
                outb.at[buf], out.at[pl.ds(r0, 256)], sout[buf])

        def wait_out(b, buf):
            r0 = (start + 2 * b) * 128
            pltpu.make_async_copy(
                outb.at[buf], out.at[pl.ds(r0, 256)], sout[buf]).wait()

        issue_in(jnp.int32(0), 0)
        issue_in(jnp.int32(1), 1)

        def pair_body(m, _):
            for sub in range(2):
                b = 2 * m + sub
                wait_in(b, sub)

                @pl.when(b >= 2)
                def _():
                    wait_out(b - 2, sub)
                transpose_block(inb.at[sub], outb.at[sub], 256)
                issue_out(b, sub)

                @pl.when(b + 2 < NBLK)
                def _():
                    issue_in(b + 2, sub)
            return 0

        lax.fori_loop(0, NBLK // 2, pair_body, 0)
        wait_out(jnp.int32(NBLK - 2), 0)
        wait_out(jnp.int32(NBLK - 1), 1)

    do_cols(t_nat, t_out)
    do_cols(c_nat, c_out)

    def do_leftover(nat, out, tj):
        pltpu.async_copy(
            nat.at[:, pl.ds(tj * 128, 128)], colb.at[:, pl.ds(0, 128)],
            sin[0]).wait()
        transpose_block(colb, outb.at[0], 128)
        pltpu.async_copy(
            outb.at[0, pl.ds(0, 128)], out.at[pl.ds(tj * 128, 128)],
            sout[0]).wait()

    def do_tail(tl, out):
        pltpu.async_copy(tl, outb.at[0, pl.ds(0, TAIL)], sin[0]).wait()
        pltpu.async_copy(
            outb.at[0, pl.ds(0, TAIL)],
            out.at[pl.ds(TCOLS * 128, TAIL)], sout[0]).wait()

    for i in range(XTRA):
        for which in range(2):
            @pl.when(wid == 24 + 2 * i + which)
            def _(i=i, which=which):
                do_leftover(t_nat if which == 0 else c_nat,
                            t_out if which == 0 else c_out,
                            CPW * NW + i)

    @pl.when(wid == 22)
    def _():
        do_tail(t_tl, t_out)

    @pl.when(wid == 23)
    def _():
        do_tail(c_tl, c_out)


def _gather_body(t_idx_hbm, c_idx_hbm, t_tab, c_tab, out_hbm,
                 t_idx_v, c_idx_v, t_rows, c_rows, out_v, sem0, sem1):
    wid = lax.axis_index("s") * NC + lax.axis_index("c")
    base = wid * BPW

    pltpu.sync_copy(t_idx_hbm.at[pl.ds(base, BPW)], t_idx_v)
    pltpu.sync_copy(c_idx_hbm.at[pl.ds(base * C, BPW * C)], c_idx_v)

    sems = (sem0, sem1)

    def issue(g, buf):
        off = g * CB
        cps = [pltpu.async_copy(
            t_tab.at[t_idx_v.at[pl.ds(off, CB)]], t_rows.at[buf], sems[buf])]
        for j in range(CB * C // MAXG):
            cps.append(pltpu.async_copy(
                c_tab.at[c_idx_v.at[pl.ds(off * C + j * MAXG, MAXG)]],
                c_rows.at[buf, pl.ds(j * MAXG, MAXG)], sems[buf]))
        return cps

    lanes = lax.broadcasted_iota(jnp.int32, (L,), 0)
    msk_last = lanes == (L - 1)

    def compute(g, buf):
        tr = t_rows.at[buf]
        cr = c_rows.at[buf]

        @plsc.parallel_loop(0, CB, unroll=4)
        def _(b):
            tv = [tr[b, pl.ds(16 * k, L)] for k in range(4)]
            ib4 = jnp.full((L,), b * C, jnp.int32)
            for c in range(C):
                s = tv[0] * cr[b * C + c, pl.ds(0, L)]
                for k in range(1, 4):
                    s = s + tv[k] * cr[b * C + c, pl.ds(16 * k, L)]
                plsc.store_scatter(out_v, [ib4 + c], plsc.cumsum(s),
                                   mask=msk_last)
        pltpu.sync_copy(out_v, out_hbm.at[pl.ds((base + g * CB) * C, CB * C)])

    pend = issue(0, 0)
    for g in range(NCHUNK):
        nxt = issue(g + 1, (g + 1) % 2) if g + 1 < NCHUNK else None
        for cp in pend:
            cp.wait()
        compute(g, g % 2)
        pend = nxt


@jax.jit
def kernel(target, context, target_table, context_table):
    if target.ndim == 2:
        target = jnp.squeeze(target, axis=1)
    mesh = plsc.VectorSubcoreMesh(core_axis_name="c", subcore_axis_name="s")
    params = pltpu.CompilerParams(needs_layout_passes=False)

    transpose = pl.kernel(
        _transpose_body,
        out_type=(jax.ShapeDtypeStruct((V, W), jnp.float32),
                  jax.ShapeDtypeStruct((V, W), jnp.float32)),
        mesh=mesh,
        scratch_types=[
            pltpu.VMEM((2, E, 256), jnp.float32),
            pltpu.VMEM((2, 256, W), jnp.float32),
            pltpu.VMEM((E, 129), jnp.float32),
            pltpu.SemaphoreType.DMA,
            pltpu.SemaphoreType.DMA,
            pltpu.SemaphoreType.DMA,
            pltpu.SemaphoreType.DMA,
        ],
        compiler_params=params,
    )
    gather = pl.kernel(
        _gather_body,
        out_type=jax.ShapeDtypeStruct((B * C,), jnp.float32),
        mesh=mesh,
        scratch_types=[
            pltpu.VMEM((BPW,), jnp.int32),
            pltpu.VMEM((BPW * C,), jnp.int32),
            pltpu.VMEM((2, CB, W), jnp.float32),
            pltpu.VMEM((2, CB * C, W), jnp.float32),
            pltpu.VMEM((CB * C,), jnp.float32),
            pltpu.SemaphoreType.DMA,
            pltpu.SemaphoreType.DMA,
        ],
        compiler_params=params,
    )
    t_tl = jnp.pad(target_table[TCOLS * 128:], ((0, 0), (0, E)))
    c_tl = jnp.pad(context_table[TCOLS * 128:], ((0, 0), (0, E)))
    t2, c2 = transpose(jnp.swapaxes(target_table, 0, 1),
                       jnp.swapaxes(context_table, 0, 1), t_tl, c_tl)
    out = gather(target.astype(jnp.int32),
                 context.astype(jnp.int32).reshape(-1), t2, c2)
    return out.reshape(B, C)
